# baseline (speedup 1.0000x reference)
import functools

import jax
import jax.numpy as jnp
from jax import lax
from jax.experimental import pallas as pl
from jax.experimental.pallas import tpu as pltpu
from jax.experimental.pallas import tpu_sc as plsc

N = 10000
D = 128
E = 320000

NC = 2
NS = 16
NW = NC * NS
CH = 128
NCHUNK = 84
KPW = NCHUNK * CH
E_PAD = NW * KPW
NP = 10240
NPW = NP // NS
PADROWS = NP - N

_sc_mesh = plsc.VectorSubcoreMesh(core_axis_name="c", subcore_axis_name="s")


def _make_deg(nchunk=NCHUNK, ch=CH, np_=NP):
    npw = np_ // NS

    @functools.partial(
        pl.kernel,
        out_type=jax.ShapeDtypeStruct((2 * np_,), jnp.float32),
        mesh=_sc_mesh,
        scratch_types=[
            pltpu.VMEM((nchunk, ch), jnp.int32),
            pltpu.VMEM((ch,), jnp.float32),
            pltpu.VMEM((npw,), jnp.float32),
            pltpu.VMEM_SHARED((np_,), jnp.float32),
        ],
    )
    def deg_kernel(dsts_hbm, out_hbm, dst_v, ones_v, zeros_v, acc0):
        cid = lax.axis_index("c")
        sid = lax.axis_index("s")
        wid = sid * NC + cid
        pltpu.sync_copy(dsts_hbm.at[wid], dst_v)
        one16 = jnp.ones((16,), jnp.float32)
        for i in range(ch // 16):
            ones_v[pl.ds(i * 16, 16)] = one16
        zero16 = jnp.zeros((16,), jnp.float32)
        for i in range(npw // 16):
            zeros_v[pl.ds(i * 16, 16)] = zero16
        pltpu.sync_copy(zeros_v, acc0.at[pl.ds(sid * npw, npw)])
        plsc.subcore_barrier()

        def body(c, carry):
            pltpu.sync_copy(ones_v, acc0.at[dst_v.at[c]], add=True)
            return carry

        lax.fori_loop(0, nchunk, body, 0)
        plsc.subcore_barrier()
        pltpu.sync_copy(acc0.at[pl.ds(sid * npw, npw)],
                        out_hbm.at[pl.ds(cid * np_ + sid * npw, npw)])

    return deg_kernel


def _make_hop(nchunk=NCHUNK, ch=CH, np_=NP, grp=6):
    npw = np_ // NS
    ngrp = nchunk // grp
    assert nchunk % grp == 0

    @functools.partial(
        pl.kernel,
        out_type=jax.ShapeDtypeStruct((2 * np_, D), jnp.float32),
        mesh=_sc_mesh,
        scratch_types=[
            pltpu.VMEM((nchunk, ch), jnp.int32),
            pltpu.VMEM((nchunk, ch), jnp.int32),
            [pltpu.VMEM((ch, D), jnp.float32) for _ in range(grp)],
            pltpu.VMEM_SHARED((np_, D), jnp.float32),
            [pltpu.SemaphoreType.DMA for _ in range(grp)],
            pltpu.SemaphoreType.DMA,
        ],
    )
    def hop_kernel(g_hbm, srcs_hbm, dsts_hbm, zeros_hbm, out_hbm,
                   src_v, dst_v, bufs, acc, gsems, ssem):
        cid = lax.axis_index("c")
        sid = lax.axis_index("s")
        wid = sid * NC + cid
        pltpu.sync_copy(srcs_hbm.at[wid], src_v)
        pltpu.sync_copy(dsts_hbm.at[wid], dst_v)
        pltpu.sync_copy(zeros_hbm, acc.at[pl.ds(sid * npw, npw)])
        plsc.subcore_barrier()

        def group(i, carry):
            base = i * grp
            for b in range(grp):
                pltpu.async_copy(g_hbm.at[src_v.at[base + b]], bufs[b], gsems[b])
            for b in range(grp):
                pltpu.make_async_copy(
                    g_hbm.at[src_v.at[base + b]], bufs[b], gsems[b]).wait()
                pltpu.async_copy(bufs[b], acc.at[dst_v.at[base + b]], ssem,
                                 add=True)
            for b in range(grp):
                pltpu.make_async_copy(
                    bufs[b], acc.at[dst_v.at[base + b]], ssem).wait()
            return carry

        lax.fori_loop(0, ngrp, group, 0)

        plsc.subcore_barrier()
        for t in range(npw // ch):
            s0 = sid * npw + t * ch
            pltpu.sync_copy(acc.at[pl.ds(s0, ch)],
                            out_hbm.at[pl.ds(cid * np_ + s0, ch)])

    return hop_kernel


def _make_hop_sync(nchunk=NCHUNK, ch=CH, np_=NP):
    npw = np_ // NS

    @functools.partial(
        pl.kernel,
        out_type=jax.ShapeDtypeStruct((2 * np_, D), jnp.float32),
        mesh=_sc_mesh,
        scratch_types=[
            pltpu.VMEM((nchunk, ch), jnp.int32),
            pltpu.VMEM((nchunk, ch), jnp.int32),
            pltpu.VMEM((ch, D), jnp.float32),
            pltpu.VMEM_SHARED((np_, D), jnp.float32),
            pltpu.SemaphoreType.DMA,
        ],
    )
    def hop_kernel(g_hbm, srcs_hbm, dsts_hbm, zeros_hbm, out_hbm,
                   src_v, dst_v, bufa, acc, sema):
        cid = lax.axis_index("c")
        sid = lax.axis_index("s")
        wid = sid * NC + cid
        pltpu.sync_copy(srcs_hbm.at[wid], src_v)
        pltpu.sync_copy(dsts_hbm.at[wid], dst_v)
        pltpu.sync_copy(zeros_hbm, acc.at[pl.ds(sid * npw, npw)])
        plsc.subcore_barrier()

        def body(c, carry):
            pltpu.async_copy(g_hbm.at[src_v.at[c]], bufa, sema)
            pltpu.make_async_copy(g_hbm.at[src_v.at[c]], bufa, sema).wait()
            pltpu.sync_copy(bufa, acc.at[dst_v.at[c]], add=True)
            return carry

        lax.fori_loop(0, nchunk, body, 0)

        plsc.subcore_barrier()
        for t in range(npw // ch):
            s0 = sid * npw + t * ch
            pltpu.sync_copy(acc.at[pl.ds(s0, ch)],
                            out_hbm.at[pl.ds(cid * np_ + s0, ch)])

    return hop_kernel


CHH = 64
NCHH = 168
NBUF = 4
GC = 12
NGRP = NCHH // GC
NPAIRG = NGRP // 2


def _make_hop_pp(np_=NP):
    npw = np_ // NS

    @functools.partial(
        pl.kernel,
        out_type=jax.ShapeDtypeStruct((2 * np_, D), jnp.float32),
        mesh=_sc_mesh,
        scratch_types=[
            pltpu.VMEM((2, GC, CHH), jnp.int32),
            pltpu.VMEM((2, GC, CHH), jnp.int32),
            [pltpu.VMEM((CHH, D), jnp.float32) for _ in range(NBUF)],
            pltpu.VMEM_SHARED((np_, D), jnp.float32),
            [pltpu.SemaphoreType.DMA for _ in range(NBUF)],
            [pltpu.SemaphoreType.DMA for _ in range(NBUF)],
            pltpu.SemaphoreType.DMA,
        ],
    )
    def hop_kernel(g_hbm, idx_hbm, zeros_hbm, out_hbm,
                   idxv0, idxv1, bufs, acc, gsems, ssems, semi):
        cid = lax.axis_index("c")
        sid = lax.axis_index("s")
        wid = sid * NC + cid
        pltpu.sync_copy(zeros_hbm, acc.at[pl.ds(sid * npw, npw)])
        pltpu.sync_copy(idx_hbm.at[wid, 0], idxv0)
        plsc.subcore_barrier()
        for k in range(NBUF - 1):
            pltpu.async_copy(g_hbm.at[idxv0.at[0, k]], bufs[k], gsems[k])

        def wait_idx(idxv):
            pltpu.make_async_copy(idx_hbm.at[wid, 0], idxv, semi).wait()

        def wait_scat(s):
            pltpu.make_async_copy(bufs[s], acc.at[idxv0.at[1, 0]], ssems[s]).wait()

        def do_group(idxcur, tail_ok, p, first):
            for j in range(GC):
                s = j % NBUF
                sn = (j + NBUF - 1) % NBUF
                if first and j == 0:
                    @pl.when(p > 0)
                    def _():
                        wait_scat(sn)
                else:
                    wait_scat(sn)
                nxt = j + NBUF - 1
                if nxt < GC:
                    pltpu.async_copy(g_hbm.at[idxcur.at[0, nxt]],
                                     bufs[sn], gsems[sn])
                else:
                    tail_ok(nxt - GC, sn)
                pltpu.make_async_copy(g_hbm.at[idxcur.at[0, j]],
                                      bufs[s], gsems[s]).wait()
                pltpu.async_copy(bufs[s], acc.at[idxcur.at[1, j]],
                                 ssems[s], add=True)

        def pair(p, carry):
            ge = 2 * p
            pltpu.async_copy(idx_hbm.at[wid, ge + 1], idxv1, semi)

            def tail_even(k, sn):
                if k == 0:
                    wait_idx(idxv1)
                pltpu.async_copy(g_hbm.at[idxv1.at[0, k]], bufs[sn], gsems[sn])

            do_group(idxv0, tail_even, p, True)

            @pl.when(p < NPAIRG - 1)
            def _():
                pltpu.async_copy(idx_hbm.at[wid, ge + 2], idxv0, semi)

            def tail_odd(k, sn):
                @pl.when(p < NPAIRG - 1)
                def _():
                    if k == 0:
                        wait_idx(idxv0)
                    pltpu.async_copy(g_hbm.at[idxv0.at[0, k]], bufs[sn], gsems[sn])

            do_group(idxv1, tail_odd, p, False)
            return carry

        lax.fori_loop(0, NPAIRG, pair, 0)
        wait_scat((GC - 1) % NBUF)

        plsc.subcore_barrier()
        for t in range(npw // CH):
            s0 = sid * npw + t * CH
            pltpu.sync_copy(acc.at[pl.ds(s0, CH)],
                            out_hbm.at[pl.ds(cid * np_ + s0, CH)])

    return hop_kernel


_deg_kernel = _make_deg()
_hop_kernel = _make_hop_sync()
_hop_kernel_pp = _make_hop_pp()


_BLK = 256


def _p1_scale(x_pad, degs):
    def body(x_ref, d_ref, g_ref, dinv_ref):
        deg = d_ref[0] + d_ref[1]
        dinv = jnp.where(deg > 0, lax.rsqrt(deg), 0.0)
        dinv_ref[...] = jnp.broadcast_to(dinv[:, None], (_BLK, 8))
        g_ref[...] = x_ref[...] * dinv[:, None]

    return pl.pallas_call(
        body,
        grid=(NP // _BLK,),
        in_specs=[
            pl.BlockSpec((_BLK, D), lambda i: (i, 0)),
            pl.BlockSpec((2, _BLK), lambda i: (0, i)),
        ],
        out_specs=[
            pl.BlockSpec((_BLK, D), lambda i: (i, 0)),
            pl.BlockSpec((_BLK, 8), lambda i: (i, 0)),
        ],
        out_shape=[
            jax.ShapeDtypeStruct((NP, D), jnp.float32),
            jax.ShapeDtypeStruct((NP, 8), jnp.float32),
        ],
    )(x_pad, degs)


def _p3_combine(s1, dinv8):
    def body(s_ref, dv_ref, g_ref):
        dv = dv_ref[:, :1]
        g_ref[...] = (s_ref[0] + s_ref[1]) * (dv * dv)

    return pl.pallas_call(
        body,
        grid=(NP // _BLK,),
        in_specs=[
            pl.BlockSpec((2, _BLK, D), lambda i: (0, i, 0)),
            pl.BlockSpec((_BLK, 8), lambda i: (i, 0)),
        ],
        out_specs=pl.BlockSpec((_BLK, D), lambda i: (i, 0)),
        out_shape=jax.ShapeDtypeStruct((NP, D), jnp.float32),
    )(s1, dinv8)


_BLK2 = 400


def _p5_linear(s2, dinv8, W, b2):
    def body(s_ref, dv_ref, w_ref, b_ref, o_ref):
        h = (s_ref[0] + s_ref[1]) * dv_ref[:, :1]
        o_ref[...] = lax.dot_general(
            h, w_ref[...], (((1,), (1,)), ((), ())),
            precision=lax.Precision.HIGHEST,
            preferred_element_type=jnp.float32) + b_ref[...]

    return pl.pallas_call(
        body,
        grid=(N // _BLK2,),
        in_specs=[
            pl.BlockSpec((2, _BLK2, D), lambda i: (0, i, 0)),
            pl.BlockSpec((_BLK2, 8), lambda i: (i, 0)),
            pl.BlockSpec((D, D), lambda i: (0, 0)),
            pl.BlockSpec((1, D), lambda i: (0, 0)),
        ],
        out_specs=pl.BlockSpec((_BLK2, D), lambda i: (i, 0)),
        out_shape=jax.ShapeDtypeStruct((N, D), jnp.float32),
    )(s2, dinv8, W, b2)


def kernel(x, edge_index, W, b):
    n_dummy = E_PAD - (E + N)
    loop_idx = jnp.arange(N, dtype=jnp.int32)
    dummy = (N + (jnp.arange(n_dummy, dtype=jnp.int32) % PADROWS))
    srcs = jnp.concatenate([edge_index[0], loop_idx, dummy]).reshape(NW, NCHUNK, CH)
    dsts = jnp.concatenate([edge_index[1], loop_idx, dummy]).reshape(NW, NCHUNK, CH)
    idxg = jnp.stack([srcs.reshape(NW, NGRP, GC, CHH),
                      dsts.reshape(NW, NGRP, GC, CHH)], axis=2)

    x_pad = jnp.zeros((NP, D), jnp.float32).at[:N].set(x)
    zerosd = jnp.zeros((NPW, D), jnp.float32)

    degs = _deg_kernel(dsts).reshape(2, NP)
    g1, dinv8 = _p1_scale(x_pad, degs)
    s1 = _hop_kernel_pp(g1, idxg, zerosd).reshape(2, NP, D)
    g2 = _p3_combine(s1, dinv8)
    s2 = _hop_kernel_pp(g2, idxg, zerosd).reshape(2, NP, D)
    return _p5_linear(s2, dinv8, W, b.reshape(1, D))


# speedup vs baseline: 1.0241x; 1.0241x over previous
import functools

import jax
import jax.numpy as jnp
from jax import lax
from jax.experimental import pallas as pl
from jax.experimental.pallas import tpu as pltpu
from jax.experimental.pallas import tpu_sc as plsc

N = 10000
D = 128
E = 320000

NC = 2
NS = 16
NW = NC * NS
CH = 128
NCHUNK = 84
KPW = NCHUNK * CH
E_PAD = NW * KPW
NP = 10240
NPW = NP // NS
PADROWS = NP - N

_sc_mesh = plsc.VectorSubcoreMesh(core_axis_name="c", subcore_axis_name="s")


def _make_deg(nchunk=NCHUNK, ch=CH, np_=NP):
    npw = np_ // NS

    @functools.partial(
        pl.kernel,
        out_type=jax.ShapeDtypeStruct((2 * np_,), jnp.float32),
        mesh=_sc_mesh,
        scratch_types=[
            pltpu.VMEM((nchunk, ch), jnp.int32),
            pltpu.VMEM((ch,), jnp.float32),
            pltpu.VMEM((npw,), jnp.float32),
            pltpu.VMEM_SHARED((np_,), jnp.float32),
        ],
    )
    def deg_kernel(dsts_hbm, out_hbm, dst_v, ones_v, zeros_v, acc0):
        cid = lax.axis_index("c")
        sid = lax.axis_index("s")
        wid = sid * NC + cid
        pltpu.sync_copy(dsts_hbm.at[wid], dst_v)
        one16 = jnp.ones((16,), jnp.float32)
        for i in range(ch // 16):
            ones_v[pl.ds(i * 16, 16)] = one16
        zero16 = jnp.zeros((16,), jnp.float32)
        for i in range(npw // 16):
            zeros_v[pl.ds(i * 16, 16)] = zero16
        pltpu.sync_copy(zeros_v, acc0.at[pl.ds(sid * npw, npw)])
        plsc.subcore_barrier()

        def body(c, carry):
            pltpu.sync_copy(ones_v, acc0.at[dst_v.at[c]], add=True)
            return carry

        lax.fori_loop(0, nchunk, body, 0)
        plsc.subcore_barrier()
        pltpu.sync_copy(acc0.at[pl.ds(sid * npw, npw)],
                        out_hbm.at[pl.ds(cid * np_ + sid * npw, npw)])

    return deg_kernel


def _make_hop(nchunk=NCHUNK, ch=CH, np_=NP, grp=6):
    npw = np_ // NS
    ngrp = nchunk // grp
    assert nchunk % grp == 0

    @functools.partial(
        pl.kernel,
        out_type=jax.ShapeDtypeStruct((2 * np_, D), jnp.float32),
        mesh=_sc_mesh,
        scratch_types=[
            pltpu.VMEM((nchunk, ch), jnp.int32),
            pltpu.VMEM((nchunk, ch), jnp.int32),
            [pltpu.VMEM((ch, D), jnp.float32) for _ in range(grp)],
            pltpu.VMEM_SHARED((np_, D), jnp.float32),
            [pltpu.SemaphoreType.DMA for _ in range(grp)],
            pltpu.SemaphoreType.DMA,
        ],
    )
    def hop_kernel(g_hbm, srcs_hbm, dsts_hbm, zeros_hbm, out_hbm,
                   src_v, dst_v, bufs, acc, gsems, ssem):
        cid = lax.axis_index("c")
        sid = lax.axis_index("s")
        wid = sid * NC + cid
        pltpu.sync_copy(srcs_hbm.at[wid], src_v)
        pltpu.sync_copy(dsts_hbm.at[wid], dst_v)
        pltpu.sync_copy(zeros_hbm, acc.at[pl.ds(sid * npw, npw)])
        plsc.subcore_barrier()

        def group(i, carry):
            base = i * grp
            for b in range(grp):
                pltpu.async_copy(g_hbm.at[src_v.at[base + b]], bufs[b], gsems[b])
            for b in range(grp):
                pltpu.make_async_copy(
                    g_hbm.at[src_v.at[base + b]], bufs[b], gsems[b]).wait()
                pltpu.async_copy(bufs[b], acc.at[dst_v.at[base + b]], ssem,
                                 add=True)
            for b in range(grp):
                pltpu.make_async_copy(
                    bufs[b], acc.at[dst_v.at[base + b]], ssem).wait()
            return carry

        lax.fori_loop(0, ngrp, group, 0)

        plsc.subcore_barrier()
        for t in range(npw // ch):
            s0 = sid * npw + t * ch
            pltpu.sync_copy(acc.at[pl.ds(s0, ch)],
                            out_hbm.at[pl.ds(cid * np_ + s0, ch)])

    return hop_kernel


def _make_hop_sync(nchunk=NCHUNK, ch=CH, np_=NP):
    npw = np_ // NS

    @functools.partial(
        pl.kernel,
        out_type=jax.ShapeDtypeStruct((2 * np_, D), jnp.float32),
        mesh=_sc_mesh,
        scratch_types=[
            pltpu.VMEM((nchunk, ch), jnp.int32),
            pltpu.VMEM((nchunk, ch), jnp.int32),
            pltpu.VMEM((ch, D), jnp.float32),
            pltpu.VMEM_SHARED((np_, D), jnp.float32),
            pltpu.SemaphoreType.DMA,
        ],
    )
    def hop_kernel(g_hbm, srcs_hbm, dsts_hbm, zeros_hbm, out_hbm,
                   src_v, dst_v, bufa, acc, sema):
        cid = lax.axis_index("c")
        sid = lax.axis_index("s")
        wid = sid * NC + cid
        pltpu.sync_copy(srcs_hbm.at[wid], src_v)
        pltpu.sync_copy(dsts_hbm.at[wid], dst_v)
        pltpu.sync_copy(zeros_hbm, acc.at[pl.ds(sid * npw, npw)])
        plsc.subcore_barrier()

        def body(c, carry):
            pltpu.async_copy(g_hbm.at[src_v.at[c]], bufa, sema)
            pltpu.make_async_copy(g_hbm.at[src_v.at[c]], bufa, sema).wait()
            pltpu.sync_copy(bufa, acc.at[dst_v.at[c]], add=True)
            return carry

        lax.fori_loop(0, nchunk, body, 0)

        plsc.subcore_barrier()
        for t in range(npw // ch):
            s0 = sid * npw + t * ch
            pltpu.sync_copy(acc.at[pl.ds(s0, ch)],
                            out_hbm.at[pl.ds(cid * np_ + s0, ch)])

    return hop_kernel


CHH = 112
NCHH = 96
NBUF = 3
GC = 6
NGRP = NCHH // GC
NPAIRG = NGRP // 2


def _make_hop_pp(np_=NP):
    npw = np_ // NS

    @functools.partial(
        pl.kernel,
        out_type=jax.ShapeDtypeStruct((2 * np_, D), jnp.float32),
        mesh=_sc_mesh,
        scratch_types=[
            pltpu.VMEM((2, GC, CHH), jnp.int32),
            pltpu.VMEM((2, GC, CHH), jnp.int32),
            [pltpu.VMEM((CHH, D), jnp.float32) for _ in range(NBUF)],
            pltpu.VMEM_SHARED((np_, D), jnp.float32),
            [pltpu.SemaphoreType.DMA for _ in range(NBUF)],
            [pltpu.SemaphoreType.DMA for _ in range(NBUF)],
            pltpu.SemaphoreType.DMA,
        ],
    )
    def hop_kernel(g_hbm, idx_hbm, zeros_hbm, out_hbm,
                   idxv0, idxv1, bufs, acc, gsems, ssems, semi):
        cid = lax.axis_index("c")
        sid = lax.axis_index("s")
        wid = sid * NC + cid
        pltpu.sync_copy(zeros_hbm, acc.at[pl.ds(sid * npw, npw)])
        pltpu.sync_copy(idx_hbm.at[wid, 0], idxv0)
        plsc.subcore_barrier()
        pltpu.async_copy(g_hbm.at[idxv0.at[0, 0]], bufs[0], gsems[0])
        pltpu.async_copy(g_hbm.at[idxv0.at[0, 1]], bufs[1], gsems[1])

        def wait_idx(idxv):
            pltpu.make_async_copy(idx_hbm.at[wid, 0], idxv, semi).wait()

        def wait_scat(s):
            pltpu.make_async_copy(bufs[s], acc.at[idxv0.at[1, 0]], ssems[s]).wait()

        def do_group(idxcur, idxnxt, tail_ok, p, first):
            for j in range(GC):
                s = j % NBUF
                sn = (j + 2) % NBUF
                if first and j == 0:
                    @pl.when(p > 0)
                    def _():
                        wait_scat(sn)
                else:
                    wait_scat(sn)
                if j < GC - 2:
                    pltpu.async_copy(g_hbm.at[idxcur.at[0, j + 2]],
                                     bufs[sn], gsems[sn])
                else:
                    if j == GC - 2:
                        tail_ok(True, sn)
                    else:
                        tail_ok(False, sn)
                pltpu.make_async_copy(g_hbm.at[idxcur.at[0, j]],
                                      bufs[s], gsems[s]).wait()
                pltpu.async_copy(bufs[s], acc.at[idxcur.at[1, j]],
                                 ssems[s], add=True)

        def pair(p, carry):
            ge = 2 * p
            pltpu.async_copy(idx_hbm.at[wid, ge + 1], idxv1, semi)

            def tail_even(is_first, sn):
                if is_first:
                    wait_idx(idxv1)
                    pltpu.async_copy(g_hbm.at[idxv1.at[0, 0]], bufs[sn], gsems[sn])
                else:
                    pltpu.async_copy(g_hbm.at[idxv1.at[0, 1]], bufs[sn], gsems[sn])

            do_group(idxv0, idxv1, tail_even, p, True)

            @pl.when(p < NPAIRG - 1)
            def _():
                pltpu.async_copy(idx_hbm.at[wid, ge + 2], idxv0, semi)

            def tail_odd(is_first, sn):
                @pl.when(p < NPAIRG - 1)
                def _():
                    if is_first:
                        wait_idx(idxv0)
                        pltpu.async_copy(g_hbm.at[idxv0.at[0, 0]], bufs[sn], gsems[sn])
                    else:
                        pltpu.async_copy(g_hbm.at[idxv0.at[0, 1]], bufs[sn], gsems[sn])

            do_group(idxv1, idxv0, tail_odd, p, False)
            return carry

        lax.fori_loop(0, NPAIRG, pair, 0)
        wait_scat((GC - 1) % NBUF)

        plsc.subcore_barrier()
        for t in range(npw // CH):
            s0 = sid * npw + t * CH
            pltpu.sync_copy(acc.at[pl.ds(s0, CH)],
                            out_hbm.at[pl.ds(cid * np_ + s0, CH)])

    return hop_kernel


_deg_kernel = _make_deg()
_hop_kernel = _make_hop_sync()
_hop_kernel_pp = _make_hop_pp()


_BLK = 256


def _p1_scale(x_pad, degs):
    def body(x_ref, d_ref, g_ref, dinv_ref):
        deg = d_ref[0] + d_ref[1]
        dinv = jnp.where(deg > 0, lax.rsqrt(deg), 0.0)
        dinv_ref[...] = jnp.broadcast_to(dinv[:, None], (_BLK, 8))
        g_ref[...] = x_ref[...] * dinv[:, None]

    return pl.pallas_call(
        body,
        grid=(NP // _BLK,),
        in_specs=[
            pl.BlockSpec((_BLK, D), lambda i: (i, 0)),
            pl.BlockSpec((2, _BLK), lambda i: (0, i)),
        ],
        out_specs=[
            pl.BlockSpec((_BLK, D), lambda i: (i, 0)),
            pl.BlockSpec((_BLK, 8), lambda i: (i, 0)),
        ],
        out_shape=[
            jax.ShapeDtypeStruct((NP, D), jnp.float32),
            jax.ShapeDtypeStruct((NP, 8), jnp.float32),
        ],
    )(x_pad, degs)


def _p3_combine(s1, dinv8):
    def body(s_ref, dv_ref, g_ref):
        dv = dv_ref[:, :1]
        g_ref[...] = (s_ref[0] + s_ref[1]) * (dv * dv)

    return pl.pallas_call(
        body,
        grid=(NP // _BLK,),
        in_specs=[
            pl.BlockSpec((2, _BLK, D), lambda i: (0, i, 0)),
            pl.BlockSpec((_BLK, 8), lambda i: (i, 0)),
        ],
        out_specs=pl.BlockSpec((_BLK, D), lambda i: (i, 0)),
        out_shape=jax.ShapeDtypeStruct((NP, D), jnp.float32),
    )(s1, dinv8)


_BLK2 = 400


def _p5_linear(s2, dinv8, W, b2):
    def body(s_ref, dv_ref, w_ref, b_ref, o_ref):
        h = (s_ref[0] + s_ref[1]) * dv_ref[:, :1]
        o_ref[...] = lax.dot_general(
            h, w_ref[...], (((1,), (1,)), ((), ())),
            precision=lax.Precision.HIGHEST,
            preferred_element_type=jnp.float32) + b_ref[...]

    return pl.pallas_call(
        body,
        grid=(N // _BLK2,),
        in_specs=[
            pl.BlockSpec((2, _BLK2, D), lambda i: (0, i, 0)),
            pl.BlockSpec((_BLK2, 8), lambda i: (i, 0)),
            pl.BlockSpec((D, D), lambda i: (0, 0)),
            pl.BlockSpec((1, D), lambda i: (0, 0)),
        ],
        out_specs=pl.BlockSpec((_BLK2, D), lambda i: (i, 0)),
        out_shape=jax.ShapeDtypeStruct((N, D), jnp.float32),
    )(s2, dinv8, W, b2)


def kernel(x, edge_index, W, b):
    n_dummy = E_PAD - (E + N)
    loop_idx = jnp.arange(N, dtype=jnp.int32)
    dummy = (N + (jnp.arange(n_dummy, dtype=jnp.int32) % PADROWS))
    srcs = jnp.concatenate([edge_index[0], loop_idx, dummy]).reshape(NW, NCHUNK, CH)
    dsts = jnp.concatenate([edge_index[1], loop_idx, dummy]).reshape(NW, NCHUNK, CH)
    idxg = jnp.stack([srcs.reshape(NW, NGRP, GC, CHH),
                      dsts.reshape(NW, NGRP, GC, CHH)], axis=2)

    x_pad = jnp.zeros((NP, D), jnp.float32).at[:N].set(x)
    zerosd = jnp.zeros((NPW, D), jnp.float32)

    degs = _deg_kernel(dsts).reshape(2, NP)
    g1, dinv8 = _p1_scale(x_pad, degs)
    s1 = _hop_kernel_pp(g1, idxg, zerosd).reshape(2, NP, D)
    g2 = _p3_combine(s1, dinv8)
    s2 = _hop_kernel_pp(g2, idxg, zerosd).reshape(2, NP, D)
    return _p5_linear(s2, dinv8, W, b.reshape(1, D))


# speedup vs baseline: 1.1909x; 1.1629x over previous
import functools

import jax
import jax.numpy as jnp
from jax import lax
from jax.experimental import pallas as pl
from jax.experimental.pallas import tpu as pltpu
from jax.experimental.pallas import tpu_sc as plsc

N = 10000
D = 128
E = 320000

NC = 2
NS = 16
NW = NC * NS
CH = 128
NCHUNK = 84
KPW = NCHUNK * CH
E_PAD = NW * KPW
NP = 10240
NPW = NP // NS
PADROWS = NP - N

_sc_mesh = plsc.VectorSubcoreMesh(core_axis_name="c", subcore_axis_name="s")


def _make_deg(nchunk=NCHUNK, ch=CH, np_=NP):
    npw = np_ // NS

    @functools.partial(
        pl.kernel,
        out_type=jax.ShapeDtypeStruct((2 * np_,), jnp.float32),
        mesh=_sc_mesh,
        scratch_types=[
            pltpu.VMEM((nchunk, ch), jnp.int32),
            pltpu.VMEM((ch,), jnp.float32),
            pltpu.VMEM((npw,), jnp.float32),
            pltpu.VMEM_SHARED((np_,), jnp.float32),
        ],
    )
    def deg_kernel(dsts_hbm, out_hbm, dst_v, ones_v, zeros_v, acc0):
        cid = lax.axis_index("c")
        sid = lax.axis_index("s")
        wid = sid * NC + cid
        pltpu.sync_copy(dsts_hbm.at[wid], dst_v)
        one16 = jnp.ones((16,), jnp.float32)
        for i in range(ch // 16):
            ones_v[pl.ds(i * 16, 16)] = one16
        zero16 = jnp.zeros((16,), jnp.float32)
        for i in range(npw // 16):
            zeros_v[pl.ds(i * 16, 16)] = zero16
        pltpu.sync_copy(zeros_v, acc0.at[pl.ds(sid * npw, npw)])
        plsc.subcore_barrier()

        def body(c, carry):
            pltpu.sync_copy(ones_v, acc0.at[dst_v.at[c]], add=True)
            return carry

        lax.fori_loop(0, nchunk, body, 0)
        plsc.subcore_barrier()
        pltpu.sync_copy(acc0.at[pl.ds(sid * npw, npw)],
                        out_hbm.at[pl.ds(cid * np_ + sid * npw, npw)])

    return deg_kernel


def _make_hop(nchunk=NCHUNK, ch=CH, np_=NP, grp=6):
    npw = np_ // NS
    ngrp = nchunk // grp
    assert nchunk % grp == 0

    @functools.partial(
        pl.kernel,
        out_type=jax.ShapeDtypeStruct((2 * np_, D), jnp.float32),
        mesh=_sc_mesh,
        scratch_types=[
            pltpu.VMEM((nchunk, ch), jnp.int32),
            pltpu.VMEM((nchunk, ch), jnp.int32),
            [pltpu.VMEM((ch, D), jnp.float32) for _ in range(grp)],
            pltpu.VMEM_SHARED((np_, D), jnp.float32),
            [pltpu.SemaphoreType.DMA for _ in range(grp)],
            pltpu.SemaphoreType.DMA,
        ],
    )
    def hop_kernel(g_hbm, srcs_hbm, dsts_hbm, zeros_hbm, out_hbm,
                   src_v, dst_v, bufs, acc, gsems, ssem):
        cid = lax.axis_index("c")
        sid = lax.axis_index("s")
        wid = sid * NC + cid
        pltpu.sync_copy(srcs_hbm.at[wid], src_v)
        pltpu.sync_copy(dsts_hbm.at[wid], dst_v)
        pltpu.sync_copy(zeros_hbm, acc.at[pl.ds(sid * npw, npw)])
        plsc.subcore_barrier()

        def group(i, carry):
            base = i * grp
            for b in range(grp):
                pltpu.async_copy(g_hbm.at[src_v.at[base + b]], bufs[b], gsems[b])
            for b in range(grp):
                pltpu.make_async_copy(
                    g_hbm.at[src_v.at[base + b]], bufs[b], gsems[b]).wait()
                pltpu.async_copy(bufs[b], acc.at[dst_v.at[base + b]], ssem,
                                 add=True)
            for b in range(grp):
                pltpu.make_async_copy(
                    bufs[b], acc.at[dst_v.at[base + b]], ssem).wait()
            return carry

        lax.fori_loop(0, ngrp, group, 0)

        plsc.subcore_barrier()
        for t in range(npw // ch):
            s0 = sid * npw + t * ch
            pltpu.sync_copy(acc.at[pl.ds(s0, ch)],
                            out_hbm.at[pl.ds(cid * np_ + s0, ch)])

    return hop_kernel


def _make_hop_sync(nchunk=NCHUNK, ch=CH, np_=NP):
    npw = np_ // NS

    @functools.partial(
        pl.kernel,
        out_type=jax.ShapeDtypeStruct((2 * np_, D), jnp.float32),
        mesh=_sc_mesh,
        scratch_types=[
            pltpu.VMEM((nchunk, ch), jnp.int32),
            pltpu.VMEM((nchunk, ch), jnp.int32),
            pltpu.VMEM((ch, D), jnp.float32),
            pltpu.VMEM_SHARED((np_, D), jnp.float32),
            pltpu.SemaphoreType.DMA,
        ],
    )
    def hop_kernel(g_hbm, srcs_hbm, dsts_hbm, zeros_hbm, out_hbm,
                   src_v, dst_v, bufa, acc, sema):
        cid = lax.axis_index("c")
        sid = lax.axis_index("s")
        wid = sid * NC + cid
        pltpu.sync_copy(srcs_hbm.at[wid], src_v)
        pltpu.sync_copy(dsts_hbm.at[wid], dst_v)
        pltpu.sync_copy(zeros_hbm, acc.at[pl.ds(sid * npw, npw)])
        plsc.subcore_barrier()

        def body(c, carry):
            pltpu.async_copy(g_hbm.at[src_v.at[c]], bufa, sema)
            pltpu.make_async_copy(g_hbm.at[src_v.at[c]], bufa, sema).wait()
            pltpu.sync_copy(bufa, acc.at[dst_v.at[c]], add=True)
            return carry

        lax.fori_loop(0, nchunk, body, 0)

        plsc.subcore_barrier()
        for t in range(npw // ch):
            s0 = sid * npw + t * ch
            pltpu.sync_copy(acc.at[pl.ds(s0, ch)],
                            out_hbm.at[pl.ds(cid * np_ + s0, ch)])

    return hop_kernel


CHH = 112
NCHH = 96
NBUF = 3
GC = 6
NGRP = NCHH // GC
NPAIRG = NGRP // 2


def _make_hop_pp(np_=NP):
    npw = np_ // NS

    @functools.partial(
        pl.kernel,
        out_type=jax.ShapeDtypeStruct((2 * np_, D), jnp.float32),
        mesh=_sc_mesh,
        scratch_types=[
            pltpu.VMEM((2, GC, CHH), jnp.int32),
            pltpu.VMEM((2, GC, CHH), jnp.int32),
            [pltpu.VMEM((CHH, D), jnp.float32) for _ in range(NBUF)],
            pltpu.VMEM_SHARED((np_, D), jnp.float32),
            [pltpu.SemaphoreType.DMA for _ in range(NBUF)],
            [pltpu.SemaphoreType.DMA for _ in range(NBUF)],
            pltpu.SemaphoreType.DMA,
        ],
    )
    def hop_kernel(g_hbm, idx_hbm, zeros_hbm, out_hbm,
                   idxv0, idxv1, bufs, acc, gsems, ssems, semi):
        cid = lax.axis_index("c")
        sid = lax.axis_index("s")
        wid = sid * NC + cid
        pltpu.sync_copy(zeros_hbm, acc.at[pl.ds(sid * npw, npw)])
        pltpu.sync_copy(idx_hbm.at[wid, 0], idxv0)
        plsc.subcore_barrier()
        pltpu.async_copy(g_hbm.at[idxv0.at[0, 0]], bufs[0], gsems[0])
        pltpu.async_copy(g_hbm.at[idxv0.at[0, 1]], bufs[1], gsems[1])

        def wait_idx(idxv):
            pltpu.make_async_copy(idx_hbm.at[wid, 0], idxv, semi).wait()

        def wait_scat(s):
            pltpu.make_async_copy(bufs[s], acc.at[idxv0.at[1, 0]], ssems[s]).wait()

        def do_group(idxcur, idxnxt, tail_ok, p, first):
            for j in range(GC):
                s = j % NBUF
                sn = (j + 2) % NBUF
                if first and j == 0:
                    @pl.when(p > 0)
                    def _():
                        wait_scat(sn)
                else:
                    wait_scat(sn)
                if j < GC - 2:
                    pltpu.async_copy(g_hbm.at[idxcur.at[0, j + 2]],
                                     bufs[sn], gsems[sn])
                else:
                    if j == GC - 2:
                        tail_ok(True, sn)
                    else:
                        tail_ok(False, sn)
                pltpu.make_async_copy(g_hbm.at[idxcur.at[0, j]],
                                      bufs[s], gsems[s]).wait()
                pltpu.async_copy(bufs[s], acc.at[idxcur.at[1, j]],
                                 ssems[s], add=True)

        def pair(p, carry):
            ge = 2 * p
            pltpu.async_copy(idx_hbm.at[wid, ge + 1], idxv1, semi)

            def tail_even(is_first, sn):
                if is_first:
                    wait_idx(idxv1)
                    pltpu.async_copy(g_hbm.at[idxv1.at[0, 0]], bufs[sn], gsems[sn])
                else:
                    pltpu.async_copy(g_hbm.at[idxv1.at[0, 1]], bufs[sn], gsems[sn])

            do_group(idxv0, idxv1, tail_even, p, True)

            @pl.when(p < NPAIRG - 1)
            def _():
                pltpu.async_copy(idx_hbm.at[wid, ge + 2], idxv0, semi)

            def tail_odd(is_first, sn):
                @pl.when(p < NPAIRG - 1)
                def _():
                    if is_first:
                        wait_idx(idxv0)
                        pltpu.async_copy(g_hbm.at[idxv0.at[0, 0]], bufs[sn], gsems[sn])
                    else:
                        pltpu.async_copy(g_hbm.at[idxv0.at[0, 1]], bufs[sn], gsems[sn])

            do_group(idxv1, idxv0, tail_odd, p, False)
            return carry

        lax.fori_loop(0, NPAIRG, pair, 0)
        wait_scat((GC - 1) % NBUF)

        plsc.subcore_barrier()
        for t in range(npw // CH):
            s0 = sid * npw + t * CH
            pltpu.sync_copy(acc.at[pl.ds(s0, CH)],
                            out_hbm.at[pl.ds(cid * np_ + s0, CH)])

    return hop_kernel


_deg_kernel = _make_deg()
_hop_kernel = _make_hop_sync()
_hop_kernel_pp = _make_hop_pp()


_BLK = 2048


def _p1_scale(x_pad, degs):
    def body(x_ref, d_ref, g_ref, dinv_ref):
        deg = d_ref[0] + d_ref[1]
        dinv = jnp.where(deg > 0, lax.rsqrt(deg), 0.0)
        dinv_ref[...] = jnp.broadcast_to(dinv[:, None], (_BLK, 8))
        g_ref[...] = x_ref[...] * dinv[:, None]

    return pl.pallas_call(
        body,
        grid=(NP // _BLK,),
        in_specs=[
            pl.BlockSpec((_BLK, D), lambda i: (i, 0)),
            pl.BlockSpec((2, _BLK), lambda i: (0, i)),
        ],
        out_specs=[
            pl.BlockSpec((_BLK, D), lambda i: (i, 0)),
            pl.BlockSpec((_BLK, 8), lambda i: (i, 0)),
        ],
        out_shape=[
            jax.ShapeDtypeStruct((NP, D), jnp.float32),
            jax.ShapeDtypeStruct((NP, 8), jnp.float32),
        ],
    )(x_pad, degs)


def _p3_combine(s1, dinv8):
    def body(s_ref, dv_ref, g_ref):
        dv = dv_ref[:, :1]
        g_ref[...] = (s_ref[0] + s_ref[1]) * (dv * dv)

    return pl.pallas_call(
        body,
        grid=(NP // _BLK,),
        in_specs=[
            pl.BlockSpec((2, _BLK, D), lambda i: (0, i, 0)),
            pl.BlockSpec((_BLK, 8), lambda i: (i, 0)),
        ],
        out_specs=pl.BlockSpec((_BLK, D), lambda i: (i, 0)),
        out_shape=jax.ShapeDtypeStruct((NP, D), jnp.float32),
    )(s1, dinv8)


_BLK2 = 2000


def _p5_linear(s2, dinv8, W, b2):
    def body(s_ref, dv_ref, w_ref, b_ref, o_ref):
        h = (s_ref[0] + s_ref[1]) * dv_ref[:, :1]
        o_ref[...] = lax.dot_general(
            h, w_ref[...], (((1,), (1,)), ((), ())),
            precision=lax.Precision.HIGHEST,
            preferred_element_type=jnp.float32) + b_ref[...]

    return pl.pallas_call(
        body,
        grid=(N // _BLK2,),
        in_specs=[
            pl.BlockSpec((2, _BLK2, D), lambda i: (0, i, 0)),
            pl.BlockSpec((_BLK2, 8), lambda i: (i, 0)),
            pl.BlockSpec((D, D), lambda i: (0, 0)),
            pl.BlockSpec((1, D), lambda i: (0, 0)),
        ],
        out_specs=pl.BlockSpec((_BLK2, D), lambda i: (i, 0)),
        out_shape=jax.ShapeDtypeStruct((N, D), jnp.float32),
    )(s2, dinv8, W, b2)


def kernel(x, edge_index, W, b):
    n_dummy = E_PAD - (E + N)
    loop_idx = jnp.arange(N, dtype=jnp.int32)
    dummy = (N + (jnp.arange(n_dummy, dtype=jnp.int32) % PADROWS))
    srcs = jnp.concatenate([edge_index[0], loop_idx, dummy]).reshape(NW, NCHUNK, CH)
    dsts = jnp.concatenate([edge_index[1], loop_idx, dummy]).reshape(NW, NCHUNK, CH)
    idxg = jnp.stack([srcs.reshape(NW, NGRP, GC, CHH),
                      dsts.reshape(NW, NGRP, GC, CHH)], axis=2)

    x_pad = jnp.zeros((NP, D), jnp.float32).at[:N].set(x)
    zerosd = jnp.zeros((NPW, D), jnp.float32)

    degs = _deg_kernel(dsts).reshape(2, NP)
    g1, dinv8 = _p1_scale(x_pad, degs)
    s1 = _hop_kernel_pp(g1, idxg, zerosd).reshape(2, NP, D)
    g2 = _p3_combine(s1, dinv8)
    s2 = _hop_kernel_pp(g2, idxg, zerosd).reshape(2, NP, D)
    return _p5_linear(s2, dinv8, W, b.reshape(1, D))
